# baseline (speedup 1.0000x reference)
import dataclasses
import functools

import jax
import jax.numpy as jnp
from jax import lax
from jax.experimental import pallas as pl
from jax.experimental.pallas import tpu as pltpu
from jax.experimental.pallas import tpu_sc as plsc

N = 10000
E = 320000
IN_DIM = 128
HID = 64
N_CLS = 20
G = 64

NC = 2
NS = 16
L = 16
NW = NC * NS
CHUNK = 128
ROWS_PER_W = 80
E_PAD = NW * ROWS_PER_W * CHUNK
N_ACC = 10240
SLAB = N_ACC // NS
NBUF = 4

def _sc_compiler_params(linear_hbm=False):
    cp = pltpu.CompilerParams()
    cp = dataclasses.replace(cp, needs_layout_passes=False)
    if linear_hbm:
        cp = dataclasses.replace(cp, use_tc_tiling_on_sc=False)
    return cp


def _mk_mesh():
    try:
        return plsc.VectorSubcoreMesh(
            core_axis_name="c", subcore_axis_name="s",
            num_cores=NC, num_subcores=NS)
    except TypeError:
        return plsc.VectorSubcoreMesh(core_axis_name="c", subcore_axis_name="s")


@jax.jit
def _sc_degree(dst_flat):
    n_idx = ROWS_PER_W * CHUNK

    @functools.partial(
        pl.kernel,
        out_type=jax.ShapeDtypeStruct((NW, N_ACC), jnp.float32),
        mesh=_mk_mesh(),
        scratch_types=[
            pltpu.VMEM((n_idx,), jnp.int32),
            pltpu.VMEM((N_ACC,), jnp.float32),
        ],
        compiler_params=_sc_compiler_params(),
    )
    def k(dst_hbm, out_hbm, idx_v, acc_v):
        wid = lax.axis_index("s") * NC + lax.axis_index("c")
        pltpu.sync_copy(dst_hbm.at[pl.ds(wid * n_idx, n_idx)], idx_v)

        zeros = jnp.zeros((L,), jnp.float32)

        @pl.loop(0, N_ACC, step=L)
        def _zero(i):
            acc_v[pl.ds(i, L)] = zeros

        ones = jnp.ones((L,), jnp.float32)

        @pl.loop(0, n_idx, step=L)
        def _hist(i):
            idx = idx_v[pl.ds(i, L)]
            plsc.addupdate_scatter(acc_v, [idx], ones)

        pltpu.sync_copy(acc_v, out_hbm.at[wid])

    return k(dst_flat)


@jax.jit
def _sc_edge_agg(hp, src2d, dst2d):
    @functools.partial(
        pl.kernel,
        out_type=jax.ShapeDtypeStruct((NC, N_ACC, HID), jnp.float32),
        mesh=_mk_mesh(),
        scratch_types=[
            pltpu.VMEM((ROWS_PER_W + NBUF, CHUNK), jnp.int32),
            pltpu.VMEM((ROWS_PER_W + NBUF, CHUNK), jnp.int32),
        ] + [pltpu.VMEM((CHUNK, HID), jnp.float32) for _ in range(NBUF)] + [
            pltpu.VMEM_SHARED((N_ACC, HID), jnp.float32),
        ] + [pltpu.SemaphoreType.DMA for _ in range(2 * NBUF)],
        compiler_params=_sc_compiler_params(linear_hbm=True),
    )
    def k(hp_hbm, src_hbm, dst_hbm, out_hbm, src_v, dst_v, *rest):
        bufs = rest[:NBUF]
        acc = rest[NBUF]
        gs = rest[NBUF + 1:2 * NBUF + 1]
        ss = rest[2 * NBUF + 1:]
        cid = lax.axis_index("c")
        sid = lax.axis_index("s")
        wid = sid * NC + cid
        base = wid * ROWS_PER_W

        pltpu.sync_copy(src_hbm.at[pl.ds(base, ROWS_PER_W)],
                        src_v.at[pl.ds(0, ROWS_PER_W)])
        pltpu.sync_copy(dst_hbm.at[pl.ds(base, ROWS_PER_W)],
                        dst_v.at[pl.ds(0, ROWS_PER_W)])

        izeros = jnp.zeros((L,), jnp.int32)

        @pl.loop(ROWS_PER_W, ROWS_PER_W + NBUF)
        def _zi(r):
            @pl.loop(0, CHUNK, step=L)
            def _zic(c):
                src_v[r, pl.ds(c, L)] = izeros

        zeros = jnp.zeros((L,), jnp.float32)
        buf0 = bufs[0]

        @pl.loop(0, CHUNK)
        def _zr(r):
            @pl.loop(0, HID, step=L)
            def _zc(c):
                buf0[r, pl.ds(c, L)] = zeros

        slab0 = sid * SLAB

        @pl.loop(0, SLAB // CHUNK)
        def _zs(t):
            pltpu.sync_copy(buf0, acc.at[pl.ds(slab0 + t * CHUNK, CHUNK)])

        plsc.subcore_barrier()

        for b in range(NBUF):
            pltpu.async_copy(hp_hbm.at[src_v.at[b]], bufs[b], gs[b])

        @pl.loop(0, ROWS_PER_W, step=NBUF)
        def _edge(j):
            for b in range(NBUF):
                pltpu.make_async_copy(hp_hbm.at[src_v.at[j + b]], bufs[b], gs[b]).wait()
                pltpu.async_copy(bufs[b], acc.at[dst_v.at[j + b]], ss[b], add=True)
            for b in range(NBUF):
                pltpu.make_async_copy(bufs[b], acc.at[dst_v.at[j + b]], ss[b]).wait()
                pltpu.async_copy(hp_hbm.at[src_v.at[j + NBUF + b]], bufs[b], gs[b])

        for b in range(NBUF):
            pltpu.make_async_copy(hp_hbm.at[src_v.at[ROWS_PER_W + b]], bufs[b], gs[b]).wait()

        plsc.subcore_barrier()

        @pl.loop(0, SLAB // CHUNK)
        def _dr(t):
            r0 = slab0 + t * CHUNK
            pltpu.sync_copy(acc.at[pl.ds(r0, CHUNK)], out_hbm.at[cid, pl.ds(r0, CHUNK)])

    return k(hp, src2d, dst2d)


def _tc1_body(x_ref, w1_ref, degpt_ref, hp_ref, dism_ref):
    ones = jnp.ones((NW, HID), jnp.float32)
    deg = jnp.dot(degpt_ref[...], ones, preferred_element_type=jnp.float32) + 1.0
    dism = lax.rsqrt(deg)
    m = jnp.dot(x_ref[...], w1_ref[...], preferred_element_type=jnp.float32)
    dism_ref[...] = dism
    hp_ref[...] = m * dism


@jax.jit
def _tc1(xp, W1, degpt):
    return pl.pallas_call(
        _tc1_body,
        out_shape=(
            jax.ShapeDtypeStruct((N_ACC, HID), jnp.float32),
            jax.ShapeDtypeStruct((N_ACC, HID), jnp.float32),
        ),
    )(xp, W1, degpt)


def _tc2_body(agg_ref, hp1_ref, dism_ref, b1_ref, w2_ref, hp2_ref):
    dism = dism_ref[...]
    z1 = dism * (agg_ref[0] + agg_ref[1] + hp1_ref[...]) + b1_ref[...]
    h1 = jnp.maximum(z1, 0.0)
    m2 = jnp.dot(h1, w2_ref[...], preferred_element_type=jnp.float32)
    hp2_ref[...] = m2 * dism


@jax.jit
def _tc2(agg1, hp1, dism, b1r, W2):
    return pl.pallas_call(
        _tc2_body,
        out_shape=jax.ShapeDtypeStruct((N_ACC, HID), jnp.float32),
    )(agg1, hp1, dism, b1r, W2)


def _tc3_body(agg_ref, hp2_ref, dism_ref, b2_ref, batch_ref, ne_ref,
              wlin_ref, blin_ref, out_ref):
    dism = dism_ref[...]
    z2 = dism * (agg_ref[0] + agg_ref[1] + hp2_ref[...]) + b2_ref[...]
    h2 = jnp.maximum(z2, 0.0)

    gids = lax.broadcasted_iota(jnp.int32, (G, N_ACC), 0)
    M = (batch_ref[...] == gids).astype(jnp.float32)
    ne_node = jnp.dot(ne_ref[...], M, preferred_element_type=jnp.float32)
    iota_n = lax.broadcasted_iota(jnp.int32, (1, N_ACC), 1).astype(jnp.float32)
    emask = (iota_n < ne_node).astype(jnp.float32)
    Me = M * emask

    ent_sum = jnp.dot(Me, h2, preferred_element_type=jnp.float32)
    all_sum = jnp.dot(M, h2, preferred_element_type=jnp.float32)
    ent_cnt = jnp.sum(Me, axis=1, keepdims=True)
    cnt = jnp.sum(M, axis=1, keepdims=True)

    ent_mean = ent_sum / (ent_cnt + 1e-6)
    all_mean = all_sum / jnp.maximum(cnt, 1.0)
    pooled = jnp.where(ent_cnt > 0, ent_mean, all_mean)

    logits = jnp.dot(pooled, wlin_ref[...], preferred_element_type=jnp.float32)
    logits = logits + blin_ref[...]
    lmax = jnp.max(logits, axis=1, keepdims=True)
    shifted = logits - lmax
    lse = jnp.log(jnp.sum(jnp.exp(shifted), axis=1, keepdims=True))
    out_ref[...] = shifted - lse


@jax.jit
def _tc3(agg2, hp2, dism, b2r, batchp, nef, Wlin, blinr):
    return pl.pallas_call(
        _tc3_body,
        out_shape=jax.ShapeDtypeStruct((G, N_CLS), jnp.float32),
    )(agg2, hp2, dism, b2r, batchp, nef, Wlin, blinr)


@jax.jit
def kernel(x, edge_index, batch, num_entity, W1, b1, W2, b2, Wlin, blin):
    pad = E_PAD - E
    fill = jnp.full((1, pad), N, jnp.int32)
    ei = jnp.concatenate([edge_index.astype(jnp.int32), jnp.tile(fill, (2, 1))], axis=1)
    src2d = ei[0].reshape(NW * ROWS_PER_W, CHUNK)
    dst2d = ei[1].reshape(NW * ROWS_PER_W, CHUNK)
    dst_flat = ei[1]

    xp = jnp.pad(x.astype(jnp.float32), ((0, N_ACC - N), (0, 0)))
    batchp = jnp.pad(batch.astype(jnp.int32), (0, N_ACC - N),
                     constant_values=G).reshape(1, N_ACC)
    nef = num_entity.astype(jnp.float32).reshape(1, G)
    b1r = b1.reshape(1, HID)
    b2r = b2.reshape(1, HID)
    blinr = blin.reshape(1, N_CLS)

    degp = _sc_degree(dst_flat)
    degpt = degp.T

    hp1, dism = _tc1(xp, W1, degpt)
    agg1 = _sc_edge_agg(hp1, src2d, dst2d)
    hp2 = _tc2(agg1, hp1, dism, b1r, W2)
    agg2 = _sc_edge_agg(hp2, src2d, dst2d)
    return _tc3(agg2, hp2, dism, b2r, batchp, nef, Wlin, blinr)


# speedup vs baseline: 3.0429x; 3.0429x over previous
import dataclasses
import functools

import jax
import jax.numpy as jnp
from jax import lax
from jax.experimental import pallas as pl
from jax.experimental.pallas import tpu as pltpu
from jax.experimental.pallas import tpu_sc as plsc

N = 10000
E = 320000
IN_DIM = 128
HID = 64
N_CLS = 20
G = 64

NC = 2
NS = 16
L = 16
NW = NC * NS
CHUNK = 128
ROWS_PER_W = 80
E_PAD = NW * ROWS_PER_W * CHUNK
N_ACC = 10240
SLAB = N_ACC // NS
NBUF = 2

def _sc_compiler_params(linear_hbm=False):
    cp = pltpu.CompilerParams()
    cp = dataclasses.replace(cp, needs_layout_passes=False)
    if linear_hbm:
        cp = dataclasses.replace(cp, use_tc_tiling_on_sc=False)
    return cp


def _mk_mesh():
    try:
        return plsc.VectorSubcoreMesh(
            core_axis_name="c", subcore_axis_name="s",
            num_cores=NC, num_subcores=NS)
    except TypeError:
        return plsc.VectorSubcoreMesh(core_axis_name="c", subcore_axis_name="s")


@jax.jit
def _sc_degree(dst_flat):
    n_idx = ROWS_PER_W * CHUNK

    @functools.partial(
        pl.kernel,
        out_type=jax.ShapeDtypeStruct((NW, N_ACC), jnp.float32),
        mesh=_mk_mesh(),
        scratch_types=[
            pltpu.VMEM((n_idx,), jnp.int32),
            pltpu.VMEM((N_ACC,), jnp.float32),
        ],
        compiler_params=_sc_compiler_params(),
    )
    def k(dst_hbm, out_hbm, idx_v, acc_v):
        wid = lax.axis_index("s") * NC + lax.axis_index("c")
        pltpu.sync_copy(dst_hbm.at[pl.ds(wid * n_idx, n_idx)], idx_v)

        zeros = jnp.zeros((L,), jnp.float32)

        @pl.loop(0, N_ACC, step=L)
        def _zero(i):
            acc_v[pl.ds(i, L)] = zeros

        ones = jnp.ones((L,), jnp.float32)

        @pl.loop(0, n_idx, step=L)
        def _hist(i):
            idx = idx_v[pl.ds(i, L)]
            plsc.addupdate_scatter(acc_v, [idx], ones)

        pltpu.sync_copy(acc_v, out_hbm.at[wid])

    return k(dst_flat)


@jax.jit
def _sc_edge_agg(hp, src2d, dst2d):
    @functools.partial(
        pl.kernel,
        out_type=jax.ShapeDtypeStruct((NC, N_ACC, HID), jnp.float32),
        mesh=_mk_mesh(),
        scratch_types=[
            pltpu.VMEM((ROWS_PER_W + NBUF, CHUNK), jnp.int32),
            pltpu.VMEM((ROWS_PER_W + NBUF, CHUNK), jnp.int32),
        ] + [pltpu.VMEM((CHUNK, HID), jnp.float32) for _ in range(NBUF)] + [
            pltpu.VMEM_SHARED((N_ACC, HID), jnp.float32),
            pltpu.VMEM_SHARED((N_ACC, HID), jnp.float32),
        ] + [pltpu.SemaphoreType.DMA for _ in range(2 * NBUF)],
        compiler_params=_sc_compiler_params(linear_hbm=True),
    )
    def k(hp_hbm, src_hbm, dst_hbm, out_hbm, src_v, dst_v, *rest):
        bufs = rest[:NBUF]
        acc = rest[NBUF]
        hps = rest[NBUF + 1]
        gs = rest[NBUF + 2:2 * NBUF + 2]
        ss = rest[2 * NBUF + 2:]
        cid = lax.axis_index("c")
        sid = lax.axis_index("s")
        wid = sid * NC + cid
        base = wid * ROWS_PER_W

        pltpu.sync_copy(src_hbm.at[pl.ds(base, ROWS_PER_W)],
                        src_v.at[pl.ds(0, ROWS_PER_W)])
        pltpu.sync_copy(dst_hbm.at[pl.ds(base, ROWS_PER_W)],
                        dst_v.at[pl.ds(0, ROWS_PER_W)])

        izeros = jnp.zeros((L,), jnp.int32)

        @pl.loop(ROWS_PER_W, ROWS_PER_W + NBUF)
        def _zi(r):
            @pl.loop(0, CHUNK, step=L)
            def _zic(c):
                src_v[r, pl.ds(c, L)] = izeros

        zeros = jnp.zeros((L,), jnp.float32)
        buf0 = bufs[0]

        @pl.loop(0, CHUNK)
        def _zr(r):
            @pl.loop(0, HID, step=L)
            def _zc(c):
                buf0[r, pl.ds(c, L)] = zeros

        slab0 = sid * SLAB

        @pl.loop(0, SLAB // CHUNK)
        def _zs(t):
            pltpu.sync_copy(buf0, acc.at[pl.ds(slab0 + t * CHUNK, CHUNK)])

        pltpu.sync_copy(hp_hbm.at[pl.ds(slab0, SLAB)], hps.at[pl.ds(slab0, SLAB)])

        plsc.subcore_barrier()

        for b in range(NBUF):
            pltpu.async_copy(hps.at[src_v.at[b]], bufs[b], gs[b])

        @pl.loop(0, ROWS_PER_W, step=NBUF)
        def _edge(j):
            for b in range(NBUF):
                pltpu.make_async_copy(hps.at[src_v.at[j + b]], bufs[b], gs[b]).wait()
                pltpu.async_copy(bufs[b], acc.at[dst_v.at[j + b]], ss[b], add=True)
            for b in range(NBUF):
                pltpu.make_async_copy(bufs[b], acc.at[dst_v.at[j + b]], ss[b]).wait()
                pltpu.async_copy(hps.at[src_v.at[j + NBUF + b]], bufs[b], gs[b])

        for b in range(NBUF):
            pltpu.make_async_copy(hps.at[src_v.at[ROWS_PER_W + b]], bufs[b], gs[b]).wait()

        plsc.subcore_barrier()

        @pl.loop(0, SLAB // CHUNK)
        def _dr(t):
            r0 = slab0 + t * CHUNK
            pltpu.sync_copy(acc.at[pl.ds(r0, CHUNK)], out_hbm.at[cid, pl.ds(r0, CHUNK)])

    return k(hp, src2d, dst2d)


def _tc1_body(x_ref, w1_ref, degpt_ref, hp_ref, dism_ref):
    ones = jnp.ones((NW, HID), jnp.float32)
    deg = jnp.dot(degpt_ref[...], ones, preferred_element_type=jnp.float32) + 1.0
    dism = lax.rsqrt(deg)
    m = jnp.dot(x_ref[...], w1_ref[...], preferred_element_type=jnp.float32)
    dism_ref[...] = dism
    hp_ref[...] = m * dism


@jax.jit
def _tc1(xp, W1, degpt):
    return pl.pallas_call(
        _tc1_body,
        out_shape=(
            jax.ShapeDtypeStruct((N_ACC, HID), jnp.float32),
            jax.ShapeDtypeStruct((N_ACC, HID), jnp.float32),
        ),
    )(xp, W1, degpt)


def _tc2_body(agg_ref, hp1_ref, dism_ref, b1_ref, w2_ref, hp2_ref):
    dism = dism_ref[...]
    z1 = dism * (agg_ref[0] + agg_ref[1] + hp1_ref[...]) + b1_ref[...]
    h1 = jnp.maximum(z1, 0.0)
    m2 = jnp.dot(h1, w2_ref[...], preferred_element_type=jnp.float32)
    hp2_ref[...] = m2 * dism


@jax.jit
def _tc2(agg1, hp1, dism, b1r, W2):
    return pl.pallas_call(
        _tc2_body,
        out_shape=jax.ShapeDtypeStruct((N_ACC, HID), jnp.float32),
    )(agg1, hp1, dism, b1r, W2)


def _tc3_body(agg_ref, hp2_ref, dism_ref, b2_ref, batch_ref, ne_ref,
              wlin_ref, blin_ref, out_ref):
    dism = dism_ref[...]
    z2 = dism * (agg_ref[0] + agg_ref[1] + hp2_ref[...]) + b2_ref[...]
    h2 = jnp.maximum(z2, 0.0)

    gids = lax.broadcasted_iota(jnp.int32, (G, N_ACC), 0)
    M = (batch_ref[...] == gids).astype(jnp.float32)
    ne_node = jnp.dot(ne_ref[...], M, preferred_element_type=jnp.float32)
    iota_n = lax.broadcasted_iota(jnp.int32, (1, N_ACC), 1).astype(jnp.float32)
    emask = (iota_n < ne_node).astype(jnp.float32)
    Me = M * emask

    ent_sum = jnp.dot(Me, h2, preferred_element_type=jnp.float32)
    all_sum = jnp.dot(M, h2, preferred_element_type=jnp.float32)
    ent_cnt = jnp.sum(Me, axis=1, keepdims=True)
    cnt = jnp.sum(M, axis=1, keepdims=True)

    ent_mean = ent_sum / (ent_cnt + 1e-6)
    all_mean = all_sum / jnp.maximum(cnt, 1.0)
    pooled = jnp.where(ent_cnt > 0, ent_mean, all_mean)

    logits = jnp.dot(pooled, wlin_ref[...], preferred_element_type=jnp.float32)
    logits = logits + blin_ref[...]
    lmax = jnp.max(logits, axis=1, keepdims=True)
    shifted = logits - lmax
    lse = jnp.log(jnp.sum(jnp.exp(shifted), axis=1, keepdims=True))
    out_ref[...] = shifted - lse


@jax.jit
def _tc3(agg2, hp2, dism, b2r, batchp, nef, Wlin, blinr):
    return pl.pallas_call(
        _tc3_body,
        out_shape=jax.ShapeDtypeStruct((G, N_CLS), jnp.float32),
    )(agg2, hp2, dism, b2r, batchp, nef, Wlin, blinr)


@jax.jit
def kernel(x, edge_index, batch, num_entity, W1, b1, W2, b2, Wlin, blin):
    pad = E_PAD - E
    fill = jnp.full((1, pad), N, jnp.int32)
    ei = jnp.concatenate([edge_index.astype(jnp.int32), jnp.tile(fill, (2, 1))], axis=1)
    src2d = ei[0].reshape(NW * ROWS_PER_W, CHUNK)
    dst2d = ei[1].reshape(NW * ROWS_PER_W, CHUNK)
    dst_flat = ei[1]

    xp = jnp.pad(x.astype(jnp.float32), ((0, N_ACC - N), (0, 0)))
    batchp = jnp.pad(batch.astype(jnp.int32), (0, N_ACC - N),
                     constant_values=G).reshape(1, N_ACC)
    nef = num_entity.astype(jnp.float32).reshape(1, G)
    b1r = b1.reshape(1, HID)
    b2r = b2.reshape(1, HID)
    blinr = blin.reshape(1, N_CLS)

    degp = _sc_degree(dst_flat)
    degpt = degp.T

    hp1, dism = _tc1(xp, W1, degpt)
    agg1 = _sc_edge_agg(hp1, src2d, dst2d)
    hp2 = _tc2(agg1, hp1, dism, b1r, W2)
    agg2 = _sc_edge_agg(hp2, src2d, dst2d)
    return _tc3(agg2, hp2, dism, b2r, batchp, nef, Wlin, blinr)


# speedup vs baseline: 3.1516x; 1.0357x over previous
import dataclasses
import functools

import jax
import jax.numpy as jnp
from jax import lax
from jax.experimental import pallas as pl
from jax.experimental.pallas import tpu as pltpu
from jax.experimental.pallas import tpu_sc as plsc

N = 10000
E = 320000
IN_DIM = 128
HID = 64
N_CLS = 20
G = 64

NC = 2
NS = 16
L = 16
NW = NC * NS
CHUNK = 128
ROWS_PER_W = 80
E_PAD = NW * ROWS_PER_W * CHUNK
N_ACC = 10240
SLAB = N_ACC // NS
NBUF = 2

def _sc_compiler_params(linear_hbm=False):
    cp = pltpu.CompilerParams()
    cp = dataclasses.replace(cp, needs_layout_passes=False)
    if linear_hbm:
        cp = dataclasses.replace(cp, use_tc_tiling_on_sc=False)
    return cp


def _mk_mesh():
    try:
        return plsc.VectorSubcoreMesh(
            core_axis_name="c", subcore_axis_name="s",
            num_cores=NC, num_subcores=NS)
    except TypeError:
        return plsc.VectorSubcoreMesh(core_axis_name="c", subcore_axis_name="s")


@jax.jit
def _sc_degree(dst_flat):
    n_idx = ROWS_PER_W * CHUNK

    @functools.partial(
        pl.kernel,
        out_type=jax.ShapeDtypeStruct((NW, N_ACC), jnp.float32),
        mesh=_mk_mesh(),
        scratch_types=[
            pltpu.VMEM((n_idx,), jnp.int32),
            pltpu.VMEM((N_ACC,), jnp.float32),
        ],
        compiler_params=_sc_compiler_params(),
    )
    def k(dst_hbm, out_hbm, idx_v, acc_v):
        wid = lax.axis_index("s") * NC + lax.axis_index("c")
        pltpu.sync_copy(dst_hbm.at[pl.ds(wid * n_idx, n_idx)], idx_v)

        zeros = jnp.zeros((L,), jnp.float32)

        @pl.loop(0, N_ACC, step=L)
        def _zero(i):
            acc_v[pl.ds(i, L)] = zeros

        ones = jnp.ones((L,), jnp.float32)

        @plsc.parallel_loop(0, n_idx, step=L, unroll=8)
        def _hist(i):
            idx = idx_v[pl.ds(i, L)]
            plsc.addupdate_scatter(acc_v, [idx], ones)

        pltpu.sync_copy(acc_v, out_hbm.at[wid])

    return k(dst_flat)


@jax.jit
def _sc_edge_agg(hp, src2d, dst2d):
    @functools.partial(
        pl.kernel,
        out_type=jax.ShapeDtypeStruct((NC, N_ACC, HID), jnp.float32),
        mesh=_mk_mesh(),
        scratch_types=[
            pltpu.VMEM((ROWS_PER_W + NBUF, CHUNK), jnp.int32),
            pltpu.VMEM((ROWS_PER_W + NBUF, CHUNK), jnp.int32),
        ] + [pltpu.VMEM((CHUNK, HID), jnp.float32) for _ in range(NBUF)] + [
            pltpu.VMEM_SHARED((N_ACC, HID), jnp.float32),
            pltpu.VMEM_SHARED((N_ACC, HID), jnp.float32),
        ] + [pltpu.SemaphoreType.DMA for _ in range(2 * NBUF)],
        compiler_params=_sc_compiler_params(linear_hbm=True),
    )
    def k(hp_hbm, src_hbm, dst_hbm, out_hbm, src_v, dst_v, *rest):
        bufs = rest[:NBUF]
        acc = rest[NBUF]
        hps = rest[NBUF + 1]
        gs = rest[NBUF + 2:2 * NBUF + 2]
        ss = rest[2 * NBUF + 2:]
        cid = lax.axis_index("c")
        sid = lax.axis_index("s")
        wid = sid * NC + cid
        base = wid * ROWS_PER_W

        slab0 = sid * SLAB

        a_src = pltpu.async_copy(src_hbm.at[pl.ds(base, ROWS_PER_W)],
                                 src_v.at[pl.ds(0, ROWS_PER_W)], gs[0])
        a_dst = pltpu.async_copy(dst_hbm.at[pl.ds(base, ROWS_PER_W)],
                                 dst_v.at[pl.ds(0, ROWS_PER_W)], gs[1])
        a_stage = pltpu.async_copy(hp_hbm.at[pl.ds(slab0, SLAB)],
                                   hps.at[pl.ds(slab0, SLAB)], ss[0])

        izeros = jnp.zeros((L,), jnp.int32)

        @pl.loop(ROWS_PER_W, ROWS_PER_W + NBUF)
        def _zi(r):
            @pl.loop(0, CHUNK, step=L)
            def _zic(c):
                src_v[r, pl.ds(c, L)] = izeros

        zeros = jnp.zeros((L,), jnp.float32)
        buf0 = bufs[0]

        @pl.loop(0, CHUNK)
        def _zr(r):
            @pl.loop(0, HID, step=L)
            def _zc(c):
                buf0[r, pl.ds(c, L)] = zeros

        for t in range(SLAB // CHUNK):
            pltpu.async_copy(buf0, acc.at[pl.ds(slab0 + t * CHUNK, CHUNK)], ss[1])

        a_src.wait()
        a_dst.wait()
        a_stage.wait()
        for t in range(SLAB // CHUNK):
            pltpu.make_async_copy(buf0, acc.at[pl.ds(slab0 + t * CHUNK, CHUNK)],
                                  ss[1]).wait()

        plsc.subcore_barrier()

        for b in range(NBUF):
            pltpu.async_copy(hps.at[src_v.at[b]], bufs[b], gs[b])

        @pl.loop(0, ROWS_PER_W, step=NBUF)
        def _edge(j):
            for b in range(NBUF):
                pltpu.make_async_copy(hps.at[src_v.at[j + b]], bufs[b], gs[b]).wait()
                pltpu.async_copy(bufs[b], acc.at[dst_v.at[j + b]], ss[b], add=True)
            for b in range(NBUF):
                pltpu.make_async_copy(bufs[b], acc.at[dst_v.at[j + b]], ss[b]).wait()
                pltpu.async_copy(hps.at[src_v.at[j + NBUF + b]], bufs[b], gs[b])

        for b in range(NBUF):
            pltpu.make_async_copy(hps.at[src_v.at[ROWS_PER_W + b]], bufs[b], gs[b]).wait()

        plsc.subcore_barrier()

        for t in range(SLAB // CHUNK):
            r0 = slab0 + t * CHUNK
            pltpu.async_copy(acc.at[pl.ds(r0, CHUNK)],
                             out_hbm.at[cid, pl.ds(r0, CHUNK)], gs[0])
        for t in range(SLAB // CHUNK):
            r0 = slab0 + t * CHUNK
            pltpu.make_async_copy(acc.at[pl.ds(r0, CHUNK)],
                                  out_hbm.at[cid, pl.ds(r0, CHUNK)], gs[0]).wait()

    return k(hp, src2d, dst2d)


def _tc1_body(x_ref, w1_ref, degpt_ref, hp_ref, dism_ref):
    ones = jnp.ones((NW, HID), jnp.float32)
    deg = jnp.dot(degpt_ref[...], ones, preferred_element_type=jnp.float32) + 1.0
    dism = lax.rsqrt(deg)
    m = jnp.dot(x_ref[...], w1_ref[...], preferred_element_type=jnp.float32)
    dism_ref[...] = dism
    hp_ref[pl.ds(0, N), :] = m * dism[:N]
    hp_ref[pl.ds(N, N_ACC - N), :] = jnp.zeros((N_ACC - N, HID), jnp.float32)


@jax.jit
def _tc1(x, W1, degpt):
    return pl.pallas_call(
        _tc1_body,
        out_shape=(
            jax.ShapeDtypeStruct((N_ACC, HID), jnp.float32),
            jax.ShapeDtypeStruct((N_ACC, HID), jnp.float32),
        ),
    )(x, W1, degpt)


def _tc2_body(agg_ref, hp1_ref, dism_ref, b1_ref, w2_ref, hp2_ref):
    dism = dism_ref[...]
    z1 = dism * (agg_ref[0] + agg_ref[1] + hp1_ref[...]) + b1_ref[...]
    h1 = jnp.maximum(z1, 0.0)
    m2 = jnp.dot(h1, w2_ref[...], preferred_element_type=jnp.float32)
    hp2_ref[...] = m2 * dism


@jax.jit
def _tc2(agg1, hp1, dism, b1r, W2):
    return pl.pallas_call(
        _tc2_body,
        out_shape=jax.ShapeDtypeStruct((N_ACC, HID), jnp.float32),
    )(agg1, hp1, dism, b1r, W2)


def _tc3_body(agg_ref, hp2_ref, dism_ref, b2_ref, batch_ref, ne_ref,
              wlin_ref, blin_ref, out_ref):
    dism = dism_ref[...]
    z2 = dism * (agg_ref[0] + agg_ref[1] + hp2_ref[...]) + b2_ref[...]
    h2 = jnp.maximum(z2, 0.0)

    gids = lax.broadcasted_iota(jnp.int32, (G, N_ACC), 0)
    M = (batch_ref[...] == gids).astype(jnp.float32)
    ne_node = jnp.dot(ne_ref[...], M, preferred_element_type=jnp.float32)
    iota_n = lax.broadcasted_iota(jnp.int32, (1, N_ACC), 1).astype(jnp.float32)
    emask = (iota_n < ne_node).astype(jnp.float32)
    Me = M * emask

    ent_sum = jnp.dot(Me, h2, preferred_element_type=jnp.float32)
    all_sum = jnp.dot(M, h2, preferred_element_type=jnp.float32)
    ent_cnt = jnp.sum(Me, axis=1, keepdims=True)
    cnt = jnp.sum(M, axis=1, keepdims=True)

    ent_mean = ent_sum / (ent_cnt + 1e-6)
    all_mean = all_sum / jnp.maximum(cnt, 1.0)
    pooled = jnp.where(ent_cnt > 0, ent_mean, all_mean)

    logits = jnp.dot(pooled, wlin_ref[...], preferred_element_type=jnp.float32)
    logits = logits + blin_ref[...]
    lmax = jnp.max(logits, axis=1, keepdims=True)
    shifted = logits - lmax
    lse = jnp.log(jnp.sum(jnp.exp(shifted), axis=1, keepdims=True))
    out_ref[...] = shifted - lse


@jax.jit
def _tc3(agg2, hp2, dism, b2r, batchp, nef, Wlin, blinr):
    return pl.pallas_call(
        _tc3_body,
        out_shape=jax.ShapeDtypeStruct((G, N_CLS), jnp.float32),
    )(agg2, hp2, dism, b2r, batchp, nef, Wlin, blinr)


@jax.jit
def kernel(x, edge_index, batch, num_entity, W1, b1, W2, b2, Wlin, blin):
    pad = E_PAD - E
    fill = jnp.full((1, pad), N, jnp.int32)
    ei = jnp.concatenate([edge_index.astype(jnp.int32), jnp.tile(fill, (2, 1))], axis=1)
    src2d = ei[0].reshape(NW * ROWS_PER_W, CHUNK)
    dst2d = ei[1].reshape(NW * ROWS_PER_W, CHUNK)
    dst_flat = ei[1]

    batchp = jnp.pad(batch.astype(jnp.int32), (0, N_ACC - N),
                     constant_values=G).reshape(1, N_ACC)
    nef = num_entity.astype(jnp.float32).reshape(1, G)
    b1r = b1.reshape(1, HID)
    b2r = b2.reshape(1, HID)
    blinr = blin.reshape(1, N_CLS)

    degp = _sc_degree(dst_flat)
    degpt = degp.T

    hp1, dism = _tc1(x.astype(jnp.float32), W1, degpt)
    agg1 = _sc_edge_agg(hp1, src2d, dst2d)
    hp2 = _tc2(agg1, hp1, dism, b1r, W2)
    agg2 = _sc_edge_agg(hp2, src2d, dst2d)
    return _tc3(agg2, hp2, dism, b2r, batchp, nef, Wlin, blinr)
